# Initial kernel scaffold; baseline (speedup 1.0000x reference)
import jax
import jax.numpy as jnp
from jax.experimental import pallas as pl


def kernel(x, pos, batch, edge_index_3rd, parent_child_pairs, negative_pairs, edge_index, W1, b1, W2, b2, W3, b3):
    raise NotImplementedError("write your pallas kernel here")


# speedup vs baseline: 2.0454x; 2.0454x over previous
import math

import jax
import jax.numpy as jnp
from jax import lax
from jax.experimental import pallas as pl
from jax.experimental.pallas import tpu as pltpu
from jax.experimental.pallas import tpu_sc as plsc

N = 10000
E = 320000
P = 320000
D = 128
NUM_RBF = 20
FS = 3
DELTA = 1.0

NC = 2
NS = 16
NW = NC * NS
EPT = E // NW
EPC = E // NC
PPT = P // NW
CE = 80
CP = 80
C2 = 2000
NPAD = 10240

_PI = float(math.pi)
_HALF_PI = float(math.pi / 2.0)


def _fsplat(v):
    return jnp.full((16,), v, dtype=jnp.float32)


def _sqrt16(s):
    i = plsc.bitcast(s, jnp.int32)
    i = jnp.int32(0x5F3759DF) - lax.shift_right_logical(i, 1)
    r = plsc.bitcast(i, jnp.float32)
    for _ in range(3):
        r = r * (_fsplat(1.5) - _fsplat(0.5) * s * r * r)
    return s * r


def _abs_atan2_16(y, x):
    a = jnp.abs(y)
    b = jnp.abs(x)
    mx = jnp.maximum(a, b)
    mn = jnp.minimum(a, b)
    t = mn / jnp.maximum(mx, _fsplat(1e-30))
    t2 = t * t
    p = _fsplat(-0.0117212)
    p = p * t2 + _fsplat(0.05265332)
    p = p * t2 + _fsplat(-0.11643287)
    p = p * t2 + _fsplat(0.19354346)
    p = p * t2 + _fsplat(-0.33262347)
    p = p * t2 + _fsplat(0.99997726)
    th = t * p
    th = jnp.where(a > b, _fsplat(_HALF_PI) - th, th)
    th = jnp.where(x < 0.0, _fsplat(_PI) - th, th)
    return th


def _geom_body(posx_hbm, posy_hbm, posz_hbm, epar_hbm, echi_hbm,
               x_hbm, ppa_hbm, ppb_hbm, nga_hbm, ngb_hbm,
               dist_out, ang_out, pos_out, neg_out,
               posx_v, posy_v, posz_v, eidx_p, eidx_c, dist_v, ang_v,
               idx_a, idx_b, rows_a, rows_b, accp_v, accn_v, sem):
    c = lax.axis_index("c")
    s = lax.axis_index("s")
    wid = s * NC + c
    lanes = lax.iota(jnp.int32, 16)

    pltpu.sync_copy(posx_hbm, posx_v)
    pltpu.sync_copy(posy_hbm, posy_v)
    pltpu.sync_copy(posz_hbm, posz_v)
    z16 = _fsplat(0.0)
    accp_v[...] = z16
    accn_v[...] = z16

    ebase = wid * EPT

    def _edge_chunk(i, carry):
        base = ebase + i * CE
        pltpu.sync_copy(epar_hbm.at[pl.ds(base, CE)], eidx_p)
        pltpu.sync_copy(echi_hbm.at[pl.ds(base, CE)], eidx_c)

        def _group(g, carry2):
            pi = eidx_p[pl.ds(g * 16, 16)]
            ci = eidx_c[pl.ds(g * 16, 16)]
            dx = plsc.load_gather(posx_v, [ci]) - plsc.load_gather(posx_v, [pi])
            dy = plsc.load_gather(posy_v, [ci]) - plsc.load_gather(posy_v, [pi])
            dz = plsc.load_gather(posz_v, [ci]) - plsc.load_gather(posz_v, [pi])
            dist_v[pl.ds(g * 16, 16)] = _sqrt16(dx * dx + dy * dy + dz * dz)
            ang_v[pl.ds(g * 16, 16)] = _abs_atan2_16(dy, dx)
            return carry2
        lax.fori_loop(0, CE // 16, _group, 0)
        pltpu.sync_copy(dist_v, dist_out.at[pl.ds(base, CE)])
        pltpu.sync_copy(ang_v, ang_out.at[pl.ds(base, CE)])
        return carry
    lax.fori_loop(0, EPT // CE, _edge_chunk, 0)

    pbase = wid * PPT

    def _pos_chunk(i, carry):
        base = pbase + i * CP
        pltpu.sync_copy(ppa_hbm.at[pl.ds(base, CP)], idx_a)
        pltpu.sync_copy(ppb_hbm.at[pl.ds(base, CP)], idx_b)
        cpa = pltpu.async_copy(x_hbm.at[idx_a], rows_a, sem)
        cpb = pltpu.async_copy(x_hbm.at[idx_b], rows_b, sem)
        cpa.wait()
        cpb.wait()

        def _pair(j, carry2):
            t = _fsplat(0.0)
            for k in range(D // 16):
                va = rows_a[j, pl.ds(k * 16, 16)]
                vb = rows_b[j, pl.ds(k * 16, 16)]
                t = t + jnp.maximum(vb - va, 0.0)
            plsc.addupdate(accp_v.at[pl.ds(0, 16)], t)
            return carry2
        lax.fori_loop(0, CP, _pair, 0)
        return carry
    lax.fori_loop(0, PPT // CP, _pos_chunk, 0)

    def _neg_chunk(i, carry):
        base = pbase + i * CP
        pltpu.sync_copy(nga_hbm.at[pl.ds(base, CP)], idx_a)
        pltpu.sync_copy(ngb_hbm.at[pl.ds(base, CP)], idx_b)
        cpa = pltpu.async_copy(x_hbm.at[idx_a], rows_a, sem)
        cpb = pltpu.async_copy(x_hbm.at[idx_b], rows_b, sem)
        cpa.wait()
        cpb.wait()

        def _grp(g, carry2):
            rows = g * 16 + lanes
            ssum = _fsplat(0.0)
            for k in range(D):
                kv = jnp.full((16,), k, jnp.int32)
                va = plsc.load_gather(rows_a, [rows, kv])
                vb = plsc.load_gather(rows_b, [rows, kv])
                df = va - vb
                ssum = ssum + df * df
            d = _sqrt16(ssum)
            plsc.addupdate(accn_v.at[pl.ds(0, 16)],
                           jnp.maximum(_fsplat(DELTA) - d, 0.0))
            return carry2
        lax.fori_loop(0, CP // 16, _grp, 0)
        return carry
    lax.fori_loop(0, PPT // CP, _neg_chunk, 0)

    pltpu.sync_copy(accp_v, pos_out.at[pl.ds(wid * 16, 16)])
    pltpu.sync_copy(accn_v, neg_out.at[pl.ds(wid * 16, 16)])


def _seg_body(dist_hbm, ang_hbm, epar_hbm, gt_out,
              eidx_p, dist_v, ang_v, acc_v, sem):
    c = lax.axis_index("c")
    s = lax.axis_index("s")

    z16 = _fsplat(0.0)

    def _zero(i, carry):
        acc_v[pl.ds(i * 16, 16)] = z16
        acc_v[pl.ds(NPAD + i * 16, 16)] = z16
        acc_v[pl.ds(2 * NPAD + i * 16, 16)] = z16
        return carry
    lax.fori_loop(0, NPAD // 16, _zero, 0)

    slot0 = s * FS
    ebase = c * EPC

    def _chunk(i, carry):
        base = ebase + i * C2
        pltpu.sync_copy(epar_hbm.at[pl.ds(base, C2)], eidx_p)
        pltpu.sync_copy(dist_hbm.at[pl.ds(base, C2)], dist_v)
        pltpu.sync_copy(ang_hbm.at[pl.ds(base, C2)], ang_v)

        def _group(g, carry2):
            pi = eidx_p[pl.ds(g * 16, 16)]
            dv = dist_v[pl.ds(g * 16, 16)]
            av = ang_v[pl.ds(g * 16, 16)]
            for j in range(FS):
                slot = slot0 + j
                is_d = jnp.full((16,), slot, jnp.int32) < NUM_RBF
                val = jnp.where(is_d, dv, av)
                cd = (10.0 / (NUM_RBF - 1)) * slot.astype(jnp.float32)
                ca = (_PI / (NUM_RBF - 1)) * (slot - NUM_RBF).astype(jnp.float32)
                ctr = jnp.where(is_d, jnp.full((16,), 1.0, jnp.float32) * cd,
                                jnp.full((16,), 1.0, jnp.float32) * ca)
                dlt = val - ctr
                ev = jnp.exp(-(dlt * dlt))
                plsc.addupdate_scatter(acc_v.at[pl.ds(0, FS * NPAD)],
                                       [pi + j * NPAD], ev)
            return carry2
        lax.fori_loop(0, C2 // 16, _group, 0)
        return carry
    lax.fori_loop(0, EPC // C2, _chunk, 0)

    out_base = ((c * NS + s) * FS) * NPAD
    pltpu.sync_copy(acc_v, gt_out.at[pl.ds(out_base, FS * NPAD)])


def _tc_body(gt_ref, pos_ref, neg_ref, b1_ref, w2_ref, b2_ref, w3t_ref,
             b3t_ref, o_ref):
    gt_t = gt_ref[0, :NUM_RBF * 2, :N] + gt_ref[1, :NUM_RBF * 2, :N]
    ssum = jnp.sum(jnp.abs(gt_t), axis=0, keepdims=True)
    gtn = gt_t / jnp.maximum(ssum, 1e-12)
    h1 = jnp.maximum(b1_ref[...], 0.0)
    h2 = jnp.maximum(jnp.dot(h1, w2_ref[...],
                             preferred_element_type=jnp.float32)
                     + b2_ref[...], 0.0)
    pr = jnp.dot(w3t_ref[...], h2.reshape(-1, 1),
                 preferred_element_type=jnp.float32) + b3t_ref[...]
    p = pr / jnp.maximum(jnp.sum(jnp.abs(pr)), 1e-12)
    emd = jnp.sum(jnp.abs(p - gtn)) / (N * NUM_RBF * 2)
    pos_loss = jnp.sum(pos_ref[...]) / P
    neg_loss = jnp.sum(neg_ref[...]) / P
    o_ref[...] = jnp.reshape(emd + pos_loss + neg_loss, (1, 1))


def kernel(x, pos, batch, edge_index_3rd, parent_child_pairs, negative_pairs,
           edge_index, W1, b1, W2, b2, W3, b3):
    posx = pos[:, 0]
    posy = pos[:, 1]
    posz = pos[:, 2]
    epar = edge_index[0]
    echi = edge_index[1]
    ppa = parent_child_pairs[:, 0]
    ppb = parent_child_pairs[:, 1]
    nga = negative_pairs[:, 0]
    ngb = negative_pairs[:, 1]

    mesh = plsc.VectorSubcoreMesh(core_axis_name="c", subcore_axis_name="s")
    geom = pl.kernel(
        _geom_body,
        out_type=[
            jax.ShapeDtypeStruct((E,), jnp.float32),
            jax.ShapeDtypeStruct((E,), jnp.float32),
            jax.ShapeDtypeStruct((NW * 16,), jnp.float32),
            jax.ShapeDtypeStruct((NW * 16,), jnp.float32),
        ],
        mesh=mesh,
        compiler_params=pltpu.CompilerParams(needs_layout_passes=False),
        scratch_types=[
            pltpu.VMEM((N,), jnp.float32),
            pltpu.VMEM((N,), jnp.float32),
            pltpu.VMEM((N,), jnp.float32),
            pltpu.VMEM((CE,), jnp.int32),
            pltpu.VMEM((CE,), jnp.int32),
            pltpu.VMEM((CE,), jnp.float32),
            pltpu.VMEM((CE,), jnp.float32),
            pltpu.VMEM((CP,), jnp.int32),
            pltpu.VMEM((CP,), jnp.int32),
            pltpu.VMEM((CP, D), jnp.float32),
            pltpu.VMEM((CP, D), jnp.float32),
            pltpu.VMEM((16,), jnp.float32),
            pltpu.VMEM((16,), jnp.float32),
            pltpu.SemaphoreType.DMA,
        ],
    )
    dist_e, ang_e, pos_parts, neg_parts = geom(
        posx, posy, posz, epar, echi, x, ppa, ppb, nga, ngb)

    seg = pl.kernel(
        _seg_body,
        out_type=jax.ShapeDtypeStruct((NC * NS * FS * NPAD,), jnp.float32),
        mesh=mesh,
        compiler_params=pltpu.CompilerParams(needs_layout_passes=False),
        scratch_types=[
            pltpu.VMEM((C2,), jnp.int32),
            pltpu.VMEM((C2,), jnp.float32),
            pltpu.VMEM((C2,), jnp.float32),
            pltpu.VMEM((FS * NPAD,), jnp.float32),
            pltpu.SemaphoreType.DMA,
        ],
    )
    gt_parts = seg(dist_e, ang_e, epar)

    out = pl.pallas_call(
        _tc_body,
        out_shape=jax.ShapeDtypeStruct((1, 1), jnp.float32),
    )(gt_parts.reshape(NC, NS * FS, NPAD), pos_parts.reshape(4, 128),
      neg_parts.reshape(4, 128),
      b1.reshape(1, -1), W2, b2.reshape(1, -1), W3.T, b3.reshape(-1, 1))
    return out.reshape(())


# speedup vs baseline: 2.5686x; 1.2558x over previous
import math

import jax
import jax.numpy as jnp
from jax import lax
from jax.experimental import pallas as pl
from jax.experimental.pallas import tpu as pltpu
from jax.experimental.pallas import tpu_sc as plsc

N = 10000
E = 320000
P = 320000
D = 128
NUM_RBF = 20
FS = 3
DELTA = 1.0

NC = 2
NS = 16
NW = NC * NS
EPT = E // NW
EPC = E // NC
PPT = P // NW
CE = 2000
CP = 400
GB = 80
C2 = 8000
NPAD = 10240

_PI = float(math.pi)
_HALF_PI = float(math.pi / 2.0)


def _fsplat(v):
    return jnp.full((16,), v, dtype=jnp.float32)


def _sqrt16(s):
    i = plsc.bitcast(s, jnp.int32)
    i = jnp.int32(0x5F3759DF) - lax.shift_right_logical(i, 1)
    r = plsc.bitcast(i, jnp.float32)
    for _ in range(3):
        r = r * (_fsplat(1.5) - _fsplat(0.5) * s * r * r)
    return s * r


def _abs_atan2_16(y, x):
    a = jnp.abs(y)
    b = jnp.abs(x)
    mx = jnp.maximum(a, b)
    mn = jnp.minimum(a, b)
    t = mn / jnp.maximum(mx, _fsplat(1e-30))
    t2 = t * t
    p = _fsplat(-0.0117212)
    p = p * t2 + _fsplat(0.05265332)
    p = p * t2 + _fsplat(-0.11643287)
    p = p * t2 + _fsplat(0.19354346)
    p = p * t2 + _fsplat(-0.33262347)
    p = p * t2 + _fsplat(0.99997726)
    th = t * p
    th = jnp.where(a > b, _fsplat(_HALF_PI) - th, th)
    th = jnp.where(x < 0.0, _fsplat(_PI) - th, th)
    return th


def _geom_body(posx_hbm, posy_hbm, posz_hbm, epar_hbm, echi_hbm,
               dist_out, ang_out,
               posx_v, posy_v, posz_v, eidx_p, eidx_c, dist_v, ang_v):
    c = lax.axis_index("c")
    s = lax.axis_index("s")
    wid = s * NC + c
    pltpu.sync_copy(posx_hbm, posx_v)
    pltpu.sync_copy(posy_hbm, posy_v)
    pltpu.sync_copy(posz_hbm, posz_v)
    ebase = wid * EPT

    def _edge_chunk(i, carry):
        base = ebase + i * CE
        pltpu.sync_copy(epar_hbm.at[pl.ds(base, CE)], eidx_p)
        pltpu.sync_copy(echi_hbm.at[pl.ds(base, CE)], eidx_c)

        def _group(g, carry2):
            pi = eidx_p[pl.ds(g * 16, 16)]
            ci = eidx_c[pl.ds(g * 16, 16)]
            dx = plsc.load_gather(posx_v, [ci]) - plsc.load_gather(posx_v, [pi])
            dy = plsc.load_gather(posy_v, [ci]) - plsc.load_gather(posy_v, [pi])
            dz = plsc.load_gather(posz_v, [ci]) - plsc.load_gather(posz_v, [pi])
            dist_v[pl.ds(g * 16, 16)] = _sqrt16(dx * dx + dy * dy + dz * dz)
            ang_v[pl.ds(g * 16, 16)] = _abs_atan2_16(dy, dx)
            return carry2
        lax.fori_loop(0, CE // 16, _group, 0)
        pltpu.sync_copy(dist_v, dist_out.at[pl.ds(base, CE)])
        pltpu.sync_copy(ang_v, ang_out.at[pl.ds(base, CE)])
        return carry
    lax.fori_loop(0, EPT // CE, _edge_chunk, 0)


def _pairs_body(x_hbm, ppa_hbm, ppb_hbm, nga_hbm, ngb_hbm,
                pos_out, neg_out,
                idx_a, idx_b, rows_a, rows_b, accp_v, accn_v, sem):
    c = lax.axis_index("c")
    s = lax.axis_index("s")
    wid = s * NC + c
    lanes = lax.iota(jnp.int32, 16)
    z16 = _fsplat(0.0)
    accp_v[...] = z16
    accn_v[...] = z16
    pbase = wid * PPT

    def _fetch(i, a_hbm, b_hbm):
        base = pbase + i * CP
        pltpu.sync_copy(a_hbm.at[pl.ds(base, CP)], idx_a)
        pltpu.sync_copy(b_hbm.at[pl.ds(base, CP)], idx_b)
        cps = []
        for j in range(CP // GB):
            cps.append(pltpu.async_copy(
                x_hbm.at[idx_a.at[pl.ds(j * GB, GB)]],
                rows_a.at[pl.ds(j * GB, GB)], sem))
            cps.append(pltpu.async_copy(
                x_hbm.at[idx_b.at[pl.ds(j * GB, GB)]],
                rows_b.at[pl.ds(j * GB, GB)], sem))
        for cp in cps:
            cp.wait()

    def _pos_chunk(i, carry):
        _fetch(i, ppa_hbm, ppb_hbm)

        def _pair(j, carry2):
            t = _fsplat(0.0)
            for k in range(D // 16):
                va = rows_a[j, pl.ds(k * 16, 16)]
                vb = rows_b[j, pl.ds(k * 16, 16)]
                t = t + jnp.maximum(vb - va, 0.0)
            plsc.addupdate(accp_v.at[pl.ds(0, 16)], t)
            return carry2
        lax.fori_loop(0, CP, _pair, 0)
        return carry
    lax.fori_loop(0, PPT // CP, _pos_chunk, 0)

    def _neg_chunk(i, carry):
        _fetch(i, nga_hbm, ngb_hbm)

        def _grp(g, carry2):
            rows = g * 16 + lanes
            ssum = _fsplat(0.0)
            for k in range(D):
                kv = jnp.full((16,), k, jnp.int32)
                va = plsc.load_gather(rows_a, [rows, kv])
                vb = plsc.load_gather(rows_b, [rows, kv])
                df = va - vb
                ssum = ssum + df * df
            dd = _sqrt16(ssum)
            plsc.addupdate(accn_v.at[pl.ds(0, 16)],
                           jnp.maximum(_fsplat(DELTA) - dd, 0.0))
            return carry2
        lax.fori_loop(0, CP // 16, _grp, 0)
        return carry
    lax.fori_loop(0, PPT // CP, _neg_chunk, 0)

    pltpu.sync_copy(accp_v, pos_out.at[pl.ds(wid * 16, 16)])
    pltpu.sync_copy(accn_v, neg_out.at[pl.ds(wid * 16, 16)])


def _seg_body(dist_hbm, ang_hbm, epar_hbm, gt_out,
              eidx_p, dist_v, ang_v, acc_v, sem):
    c = lax.axis_index("c")
    s = lax.axis_index("s")
    z16 = _fsplat(0.0)

    def _zero(i, carry):
        acc_v[pl.ds(i * 16, 16)] = z16
        acc_v[pl.ds(NPAD + i * 16, 16)] = z16
        acc_v[pl.ds(2 * NPAD + i * 16, 16)] = z16
        return carry
    lax.fori_loop(0, NPAD // 16, _zero, 0)

    slot0 = s * FS
    ebase = c * EPC

    def _chunk(i, carry):
        base = ebase + i * C2
        pltpu.sync_copy(epar_hbm.at[pl.ds(base, C2)], eidx_p)
        pltpu.sync_copy(dist_hbm.at[pl.ds(base, C2)], dist_v)
        pltpu.sync_copy(ang_hbm.at[pl.ds(base, C2)], ang_v)

        def _group(g, carry2):
            pi = eidx_p[pl.ds(g * 16, 16)]
            dv = dist_v[pl.ds(g * 16, 16)]
            av = ang_v[pl.ds(g * 16, 16)]
            for j in range(FS):
                slot = slot0 + j
                is_d = jnp.full((16,), slot, jnp.int32) < NUM_RBF
                val = jnp.where(is_d, dv, av)
                cd = (10.0 / (NUM_RBF - 1)) * slot.astype(jnp.float32)
                ca = (_PI / (NUM_RBF - 1)) * (slot - NUM_RBF).astype(jnp.float32)
                ctr = jnp.where(is_d, jnp.full((16,), 1.0, jnp.float32) * cd,
                                jnp.full((16,), 1.0, jnp.float32) * ca)
                dlt = val - ctr
                ev = jnp.exp(-(dlt * dlt))
                plsc.addupdate_scatter(acc_v.at[pl.ds(0, FS * NPAD)],
                                       [pi + j * NPAD], ev)
            return carry2
        lax.fori_loop(0, C2 // 16, _group, 0)
        return carry
    lax.fori_loop(0, EPC // C2, _chunk, 0)

    out_base = ((c * NS + s) * FS) * NPAD
    pltpu.sync_copy(acc_v, gt_out.at[pl.ds(out_base, FS * NPAD)])


def _tc_body(gt_ref, pos_ref, neg_ref, b1_ref, w2_ref, b2_ref, w3t_ref,
             b3t_ref, o_ref):
    gt_t = gt_ref[0, :NUM_RBF * 2, :N] + gt_ref[1, :NUM_RBF * 2, :N]
    ssum = jnp.sum(jnp.abs(gt_t), axis=0, keepdims=True)
    gtn = gt_t / jnp.maximum(ssum, 1e-12)
    h1 = jnp.maximum(b1_ref[...], 0.0)
    h2 = jnp.maximum(jnp.dot(h1, w2_ref[...],
                             preferred_element_type=jnp.float32)
                     + b2_ref[...], 0.0)
    pr = jnp.dot(w3t_ref[...], h2.reshape(-1, 1),
                 preferred_element_type=jnp.float32) + b3t_ref[...]
    p = pr / jnp.maximum(jnp.sum(jnp.abs(pr)), 1e-12)
    emd = jnp.sum(jnp.abs(p - gtn)) / (N * NUM_RBF * 2)
    pos_loss = jnp.sum(pos_ref[...]) / P
    neg_loss = jnp.sum(neg_ref[...]) / P
    o_ref[...] = jnp.reshape(emd + pos_loss + neg_loss, (1, 1))


def kernel(x, pos, batch, edge_index_3rd, parent_child_pairs, negative_pairs,
           edge_index, W1, b1, W2, b2, W3, b3):
    posx = pos[:, 0]
    posy = pos[:, 1]
    posz = pos[:, 2]
    epar = edge_index[0]
    echi = edge_index[1]
    ppa = parent_child_pairs[:, 0]
    ppb = parent_child_pairs[:, 1]
    nga = negative_pairs[:, 0]
    ngb = negative_pairs[:, 1]

    mesh = plsc.VectorSubcoreMesh(core_axis_name="c", subcore_axis_name="s")
    params = pltpu.CompilerParams(needs_layout_passes=False)

    geom = pl.kernel(
        _geom_body,
        out_type=[
            jax.ShapeDtypeStruct((E,), jnp.float32),
            jax.ShapeDtypeStruct((E,), jnp.float32),
        ],
        mesh=mesh,
        compiler_params=params,
        scratch_types=[
            pltpu.VMEM((N,), jnp.float32),
            pltpu.VMEM((N,), jnp.float32),
            pltpu.VMEM((N,), jnp.float32),
            pltpu.VMEM((CE,), jnp.int32),
            pltpu.VMEM((CE,), jnp.int32),
            pltpu.VMEM((CE,), jnp.float32),
            pltpu.VMEM((CE,), jnp.float32),
        ],
    )
    dist_e, ang_e = geom(posx, posy, posz, epar, echi)

    pairs = pl.kernel(
        _pairs_body,
        out_type=[
            jax.ShapeDtypeStruct((NW * 16,), jnp.float32),
            jax.ShapeDtypeStruct((NW * 16,), jnp.float32),
        ],
        mesh=mesh,
        compiler_params=params,
        scratch_types=[
            pltpu.VMEM((CP,), jnp.int32),
            pltpu.VMEM((CP,), jnp.int32),
            pltpu.VMEM((CP, D), jnp.float32),
            pltpu.VMEM((CP, D), jnp.float32),
            pltpu.VMEM((16,), jnp.float32),
            pltpu.VMEM((16,), jnp.float32),
            pltpu.SemaphoreType.DMA,
        ],
    )
    pos_parts, neg_parts = pairs(x, ppa, ppb, nga, ngb)

    seg = pl.kernel(
        _seg_body,
        out_type=jax.ShapeDtypeStruct((NC * NS * FS * NPAD,), jnp.float32),
        mesh=mesh,
        compiler_params=params,
        scratch_types=[
            pltpu.VMEM((C2,), jnp.int32),
            pltpu.VMEM((C2,), jnp.float32),
            pltpu.VMEM((C2,), jnp.float32),
            pltpu.VMEM((FS * NPAD,), jnp.float32),
            pltpu.SemaphoreType.DMA,
        ],
    )
    gt_parts = seg(dist_e, ang_e, epar)

    out = pl.pallas_call(
        _tc_body,
        out_shape=jax.ShapeDtypeStruct((1, 1), jnp.float32),
    )(gt_parts.reshape(NC, NS * FS, NPAD), pos_parts.reshape(4, 128),
      neg_parts.reshape(4, 128),
      b1.reshape(1, -1), W2, b2.reshape(1, -1), W3.T, b3.reshape(-1, 1))
    return out.reshape(())


# speedup vs baseline: 2.9312x; 1.1412x over previous
import math

import jax
import jax.numpy as jnp
from jax import lax
from jax.experimental import pallas as pl
from jax.experimental.pallas import tpu as pltpu
from jax.experimental.pallas import tpu_sc as plsc

N = 10000
E = 320000
P = 320000
D = 128
NUM_RBF = 20
FS = 3
DELTA = 1.0

NC = 2
NS = 16
NW = NC * NS
EPT = E // NW
EPC = E // NC
PPT = P // NW
CE = 2000
CP = 80
C2 = 8000
NPAD = 10240

_PI = float(math.pi)
_HALF_PI = float(math.pi / 2.0)


def _fsplat(v):
    return jnp.full((16,), v, dtype=jnp.float32)


def _sqrt16(s):
    i = plsc.bitcast(s, jnp.int32)
    i = jnp.int32(0x5F3759DF) - lax.shift_right_logical(i, 1)
    r = plsc.bitcast(i, jnp.float32)
    for _ in range(3):
        r = r * (_fsplat(1.5) - _fsplat(0.5) * s * r * r)
    return s * r


def _abs_atan2_16(y, x):
    a = jnp.abs(y)
    b = jnp.abs(x)
    mx = jnp.maximum(a, b)
    mn = jnp.minimum(a, b)
    t = mn / jnp.maximum(mx, _fsplat(1e-30))
    t2 = t * t
    p = _fsplat(-0.0117212)
    p = p * t2 + _fsplat(0.05265332)
    p = p * t2 + _fsplat(-0.11643287)
    p = p * t2 + _fsplat(0.19354346)
    p = p * t2 + _fsplat(-0.33262347)
    p = p * t2 + _fsplat(0.99997726)
    th = t * p
    th = jnp.where(a > b, _fsplat(_HALF_PI) - th, th)
    th = jnp.where(x < 0.0, _fsplat(_PI) - th, th)
    return th


def _geom_body(posx_hbm, posy_hbm, posz_hbm, epar_hbm, echi_hbm,
               dist_out, ang_out,
               posx_v, posy_v, posz_v, eidx_p, eidx_c, dist_v, ang_v):
    c = lax.axis_index("c")
    s = lax.axis_index("s")
    wid = s * NC + c
    pltpu.sync_copy(posx_hbm, posx_v)
    pltpu.sync_copy(posy_hbm, posy_v)
    pltpu.sync_copy(posz_hbm, posz_v)
    ebase = wid * EPT

    def _edge_chunk(i, carry):
        base = ebase + i * CE
        pltpu.sync_copy(epar_hbm.at[pl.ds(base, CE)], eidx_p)
        pltpu.sync_copy(echi_hbm.at[pl.ds(base, CE)], eidx_c)

        def _group(g, carry2):
            pi = eidx_p[pl.ds(g * 16, 16)]
            ci = eidx_c[pl.ds(g * 16, 16)]
            dx = plsc.load_gather(posx_v, [ci]) - plsc.load_gather(posx_v, [pi])
            dy = plsc.load_gather(posy_v, [ci]) - plsc.load_gather(posy_v, [pi])
            dz = plsc.load_gather(posz_v, [ci]) - plsc.load_gather(posz_v, [pi])
            dist_v[pl.ds(g * 16, 16)] = _sqrt16(dx * dx + dy * dy + dz * dz)
            ang_v[pl.ds(g * 16, 16)] = _abs_atan2_16(dy, dx)
            return carry2
        lax.fori_loop(0, CE // 16, _group, 0)
        pltpu.sync_copy(dist_v, dist_out.at[pl.ds(base, CE)])
        pltpu.sync_copy(ang_v, ang_out.at[pl.ds(base, CE)])
        return carry
    lax.fori_loop(0, EPT // CE, _edge_chunk, 0)


def _pairs_body(x_hbm, ppa_hbm, ppb_hbm, nga_hbm, ngb_hbm,
                pos_out, neg_out,
                idxa_f, idxb_f, rows_a0, rows_b0, rows_a1, rows_b1,
                accp_v, accn_v, sem):
    c = lax.axis_index("c")
    s = lax.axis_index("s")
    wid = s * NC + c
    lanes = lax.iota(jnp.int32, 16)
    z16 = _fsplat(0.0)
    accp_v[...] = z16
    accn_v[...] = z16
    pbase = wid * PPT
    NCH = PPT // CP
    rows = ((rows_a0, rows_b0), (rows_a1, rows_b1))

    def _issue(i, par):
        ra, rb = rows[par]
        pltpu.async_copy(x_hbm.at[idxa_f.at[pl.ds(i * CP, CP)]], ra, sem)
        pltpu.async_copy(x_hbm.at[idxb_f.at[pl.ds(i * CP, CP)]], rb, sem)

    def _drain(par):
        ra, rb = rows[par]
        pltpu.make_async_copy(x_hbm.at[idxa_f.at[pl.ds(0, CP)]], ra, sem).wait()
        pltpu.make_async_copy(x_hbm.at[idxb_f.at[pl.ds(0, CP)]], rb, sem).wait()

    def _phase(a_hbm, b_hbm, compute):
        pltpu.sync_copy(a_hbm.at[pl.ds(pbase, PPT)], idxa_f)
        pltpu.sync_copy(b_hbm.at[pl.ds(pbase, PPT)], idxb_f)
        _issue(0, 0)

        def _body(t, carry):
            _issue(2 * t + 1, 1)
            _drain(0)
            compute(0)
            _issue(2 * t + 2, 0)
            _drain(1)
            compute(1)
            return carry
        lax.fori_loop(0, (NCH - 1) // 2, _body, 0)
        _drain(0)
        compute(0)

    def _compute_pos(par):
        ra, rb = rows[par]

        def _pair(j, carry2):
            t = _fsplat(0.0)
            for k in range(D // 16):
                va = ra[j, pl.ds(k * 16, 16)]
                vb = rb[j, pl.ds(k * 16, 16)]
                t = t + jnp.maximum(vb - va, 0.0)
            plsc.addupdate(accp_v.at[pl.ds(0, 16)], t)
            return carry2
        lax.fori_loop(0, CP, _pair, 0)

    def _compute_neg(par):
        ra, rb = rows[par]

        def _grp(g, carry2):
            rr = g * 16 + lanes
            ssum = _fsplat(0.0)
            for k in range(D):
                kv = jnp.full((16,), k, jnp.int32)
                va = plsc.load_gather(ra, [rr, kv])
                vb = plsc.load_gather(rb, [rr, kv])
                df = va - vb
                ssum = ssum + df * df
            dd = _sqrt16(ssum)
            plsc.addupdate(accn_v.at[pl.ds(0, 16)],
                           jnp.maximum(_fsplat(DELTA) - dd, 0.0))
            return carry2
        lax.fori_loop(0, CP // 16, _grp, 0)

    _phase(ppa_hbm, ppb_hbm, _compute_pos)
    _phase(nga_hbm, ngb_hbm, _compute_neg)

    pltpu.sync_copy(accp_v, pos_out.at[pl.ds(wid * 16, 16)])
    pltpu.sync_copy(accn_v, neg_out.at[pl.ds(wid * 16, 16)])


def _seg_body(dist_hbm, ang_hbm, epar_hbm, gt_out,
              eidx_p, dist_v, ang_v, acc_v, sem):
    c = lax.axis_index("c")
    s = lax.axis_index("s")
    z16 = _fsplat(0.0)

    def _zero(i, carry):
        acc_v[pl.ds(i * 16, 16)] = z16
        acc_v[pl.ds(NPAD + i * 16, 16)] = z16
        acc_v[pl.ds(2 * NPAD + i * 16, 16)] = z16
        return carry
    lax.fori_loop(0, NPAD // 16, _zero, 0)

    slot0 = s * FS
    ebase = c * EPC

    def _chunk(i, carry):
        base = ebase + i * C2
        pltpu.sync_copy(epar_hbm.at[pl.ds(base, C2)], eidx_p)
        pltpu.sync_copy(dist_hbm.at[pl.ds(base, C2)], dist_v)
        pltpu.sync_copy(ang_hbm.at[pl.ds(base, C2)], ang_v)

        def _group(g, carry2):
            pi = eidx_p[pl.ds(g * 16, 16)]
            dv = dist_v[pl.ds(g * 16, 16)]
            av = ang_v[pl.ds(g * 16, 16)]
            for j in range(FS):
                slot = slot0 + j
                is_d = jnp.full((16,), slot, jnp.int32) < NUM_RBF
                val = jnp.where(is_d, dv, av)
                cd = (10.0 / (NUM_RBF - 1)) * slot.astype(jnp.float32)
                ca = (_PI / (NUM_RBF - 1)) * (slot - NUM_RBF).astype(jnp.float32)
                ctr = jnp.where(is_d, jnp.full((16,), 1.0, jnp.float32) * cd,
                                jnp.full((16,), 1.0, jnp.float32) * ca)
                dlt = val - ctr
                ev = jnp.exp(-(dlt * dlt))
                plsc.addupdate_scatter(acc_v.at[pl.ds(0, FS * NPAD)],
                                       [pi + j * NPAD], ev)
            return carry2
        lax.fori_loop(0, C2 // 16, _group, 0)
        return carry
    lax.fori_loop(0, EPC // C2, _chunk, 0)

    out_base = ((c * NS + s) * FS) * NPAD
    pltpu.sync_copy(acc_v, gt_out.at[pl.ds(out_base, FS * NPAD)])


def _tc_body(gt_ref, pos_ref, neg_ref, b1_ref, w2_ref, b2_ref, w3t_ref,
             b3t_ref, o_ref):
    gt_t = gt_ref[0, :NUM_RBF * 2, :N] + gt_ref[1, :NUM_RBF * 2, :N]
    ssum = jnp.sum(jnp.abs(gt_t), axis=0, keepdims=True)
    gtn = gt_t / jnp.maximum(ssum, 1e-12)
    h1 = jnp.maximum(b1_ref[...], 0.0)
    h2 = jnp.maximum(jnp.dot(h1, w2_ref[...],
                             preferred_element_type=jnp.float32)
                     + b2_ref[...], 0.0)
    pr = jnp.dot(w3t_ref[...], h2.reshape(-1, 1),
                 preferred_element_type=jnp.float32) + b3t_ref[...]
    p = pr / jnp.maximum(jnp.sum(jnp.abs(pr)), 1e-12)
    emd = jnp.sum(jnp.abs(p - gtn)) / (N * NUM_RBF * 2)
    pos_loss = jnp.sum(pos_ref[...]) / P
    neg_loss = jnp.sum(neg_ref[...]) / P
    o_ref[...] = jnp.reshape(emd + pos_loss + neg_loss, (1, 1))


def kernel(x, pos, batch, edge_index_3rd, parent_child_pairs, negative_pairs,
           edge_index, W1, b1, W2, b2, W3, b3):
    posx = pos[:, 0]
    posy = pos[:, 1]
    posz = pos[:, 2]
    epar = edge_index[0]
    echi = edge_index[1]
    ppa = parent_child_pairs[:, 0]
    ppb = parent_child_pairs[:, 1]
    nga = negative_pairs[:, 0]
    ngb = negative_pairs[:, 1]

    mesh = plsc.VectorSubcoreMesh(core_axis_name="c", subcore_axis_name="s")
    params = pltpu.CompilerParams(needs_layout_passes=False)

    geom = pl.kernel(
        _geom_body,
        out_type=[
            jax.ShapeDtypeStruct((E,), jnp.float32),
            jax.ShapeDtypeStruct((E,), jnp.float32),
        ],
        mesh=mesh,
        compiler_params=params,
        scratch_types=[
            pltpu.VMEM((N,), jnp.float32),
            pltpu.VMEM((N,), jnp.float32),
            pltpu.VMEM((N,), jnp.float32),
            pltpu.VMEM((CE,), jnp.int32),
            pltpu.VMEM((CE,), jnp.int32),
            pltpu.VMEM((CE,), jnp.float32),
            pltpu.VMEM((CE,), jnp.float32),
        ],
    )
    dist_e, ang_e = geom(posx, posy, posz, epar, echi)

    pairs = pl.kernel(
        _pairs_body,
        out_type=[
            jax.ShapeDtypeStruct((NW * 16,), jnp.float32),
            jax.ShapeDtypeStruct((NW * 16,), jnp.float32),
        ],
        mesh=mesh,
        compiler_params=params,
        scratch_types=[
            pltpu.VMEM((PPT,), jnp.int32),
            pltpu.VMEM((PPT,), jnp.int32),
            pltpu.VMEM((CP, D), jnp.float32),
            pltpu.VMEM((CP, D), jnp.float32),
            pltpu.VMEM((CP, D), jnp.float32),
            pltpu.VMEM((CP, D), jnp.float32),
            pltpu.VMEM((16,), jnp.float32),
            pltpu.VMEM((16,), jnp.float32),
            pltpu.SemaphoreType.DMA,
        ],
    )
    pos_parts, neg_parts = pairs(x, ppa, ppb, nga, ngb)

    seg = pl.kernel(
        _seg_body,
        out_type=jax.ShapeDtypeStruct((NC * NS * FS * NPAD,), jnp.float32),
        mesh=mesh,
        compiler_params=params,
        scratch_types=[
            pltpu.VMEM((C2,), jnp.int32),
            pltpu.VMEM((C2,), jnp.float32),
            pltpu.VMEM((C2,), jnp.float32),
            pltpu.VMEM((FS * NPAD,), jnp.float32),
            pltpu.SemaphoreType.DMA,
        ],
    )
    gt_parts = seg(dist_e, ang_e, epar)

    out = pl.pallas_call(
        _tc_body,
        out_shape=jax.ShapeDtypeStruct((1, 1), jnp.float32),
    )(gt_parts.reshape(NC, NS * FS, NPAD), pos_parts.reshape(4, 128),
      neg_parts.reshape(4, 128),
      b1.reshape(1, -1), W2, b2.reshape(1, -1), W3.T, b3.reshape(-1, 1))
    return out.reshape(())


# speedup vs baseline: 2.9732x; 1.0143x over previous
import math

import jax
import jax.numpy as jnp
from jax import lax
from jax.experimental import pallas as pl
from jax.experimental.pallas import tpu as pltpu
from jax.experimental.pallas import tpu_sc as plsc

N = 10000
E = 320000
P = 320000
D = 128
NUM_RBF = 20
FS = 3
DELTA = 1.0

NC = 2
NS = 16
NW = NC * NS
EPT = E // NW
EPC = E // NC
PPT = P // NW
CE = 2000
CP = 80
C2 = 16000
NPAD = 10240

_PI = float(math.pi)
_HALF_PI = float(math.pi / 2.0)


def _fsplat(v):
    return jnp.full((16,), v, dtype=jnp.float32)


def _sqrt16(s):
    i = plsc.bitcast(s, jnp.int32)
    i = jnp.int32(0x5F3759DF) - lax.shift_right_logical(i, 1)
    r = plsc.bitcast(i, jnp.float32)
    for _ in range(3):
        r = r * (_fsplat(1.5) - _fsplat(0.5) * s * r * r)
    return s * r


def _abs_atan2_16(y, x):
    a = jnp.abs(y)
    b = jnp.abs(x)
    mx = jnp.maximum(a, b)
    mn = jnp.minimum(a, b)
    t = mn / jnp.maximum(mx, _fsplat(1e-30))
    t2 = t * t
    p = _fsplat(-0.0117212)
    p = p * t2 + _fsplat(0.05265332)
    p = p * t2 + _fsplat(-0.11643287)
    p = p * t2 + _fsplat(0.19354346)
    p = p * t2 + _fsplat(-0.33262347)
    p = p * t2 + _fsplat(0.99997726)
    th = t * p
    th = jnp.where(a > b, _fsplat(_HALF_PI) - th, th)
    th = jnp.where(x < 0.0, _fsplat(_PI) - th, th)
    return th


def _geom_body(posx_hbm, posy_hbm, posz_hbm, epar_hbm, echi_hbm,
               dist_out, ang_out,
               posx_v, posy_v, posz_v, eidx_p, eidx_c, dist_v, ang_v):
    c = lax.axis_index("c")
    s = lax.axis_index("s")
    wid = s * NC + c
    pltpu.sync_copy(posx_hbm, posx_v)
    pltpu.sync_copy(posy_hbm, posy_v)
    pltpu.sync_copy(posz_hbm, posz_v)
    ebase = wid * EPT

    def _edge_chunk(i, carry):
        base = ebase + i * CE
        pltpu.sync_copy(epar_hbm.at[pl.ds(base, CE)], eidx_p)
        pltpu.sync_copy(echi_hbm.at[pl.ds(base, CE)], eidx_c)

        def _group(g, carry2):
            pi = eidx_p[pl.ds(g * 16, 16)]
            ci = eidx_c[pl.ds(g * 16, 16)]
            dx = plsc.load_gather(posx_v, [ci]) - plsc.load_gather(posx_v, [pi])
            dy = plsc.load_gather(posy_v, [ci]) - plsc.load_gather(posy_v, [pi])
            dz = plsc.load_gather(posz_v, [ci]) - plsc.load_gather(posz_v, [pi])
            dist_v[pl.ds(g * 16, 16)] = _sqrt16(dx * dx + dy * dy + dz * dz)
            ang_v[pl.ds(g * 16, 16)] = _abs_atan2_16(dy, dx)
            return carry2
        lax.fori_loop(0, CE // 16, _group, 0)
        pltpu.sync_copy(dist_v, dist_out.at[pl.ds(base, CE)])
        pltpu.sync_copy(ang_v, ang_out.at[pl.ds(base, CE)])
        return carry
    lax.fori_loop(0, EPT // CE, _edge_chunk, 0)


def _pairs_body(x_hbm, ppa_hbm, ppb_hbm, nga_hbm, ngb_hbm,
                pos_out, neg_out,
                idxa_f, idxb_f, rows_a0, rows_b0, rows_a1, rows_b1,
                accp_v, accn_v, sem):
    c = lax.axis_index("c")
    s = lax.axis_index("s")
    wid = s * NC + c
    lanes = lax.iota(jnp.int32, 16)
    z16 = _fsplat(0.0)
    accp_v[...] = z16
    accn_v[...] = z16
    pbase = wid * PPT
    NCH = PPT // CP
    rows = ((rows_a0, rows_b0), (rows_a1, rows_b1))

    def _issue(i, par):
        ra, rb = rows[par]
        pltpu.async_copy(x_hbm.at[idxa_f.at[pl.ds(i * CP, CP)]], ra, sem)
        pltpu.async_copy(x_hbm.at[idxb_f.at[pl.ds(i * CP, CP)]], rb, sem)

    def _drain(par):
        ra, rb = rows[par]
        pltpu.make_async_copy(x_hbm.at[idxa_f.at[pl.ds(0, CP)]], ra, sem).wait()
        pltpu.make_async_copy(x_hbm.at[idxb_f.at[pl.ds(0, CP)]], rb, sem).wait()

    def _phase(a_hbm, b_hbm, compute):
        pltpu.sync_copy(a_hbm.at[pl.ds(pbase, PPT)], idxa_f)
        pltpu.sync_copy(b_hbm.at[pl.ds(pbase, PPT)], idxb_f)
        _issue(0, 0)

        def _body(t, carry):
            _issue(2 * t + 1, 1)
            _drain(0)
            compute(0)
            _issue(2 * t + 2, 0)
            _drain(1)
            compute(1)
            return carry
        lax.fori_loop(0, (NCH - 1) // 2, _body, 0)
        _drain(0)
        compute(0)

    def _compute_pos(par):
        ra, rb = rows[par]

        def _pair(j, carry2):
            t = _fsplat(0.0)
            for k in range(D // 16):
                va = ra[j, pl.ds(k * 16, 16)]
                vb = rb[j, pl.ds(k * 16, 16)]
                t = t + jnp.maximum(vb - va, 0.0)
            plsc.addupdate(accp_v.at[pl.ds(0, 16)], t)
            return carry2
        lax.fori_loop(0, CP, _pair, 0)

    def _compute_neg(par):
        ra, rb = rows[par]

        def _grp(g, carry2):
            rr = g * 16 + lanes
            ssum = _fsplat(0.0)
            for k in range(D):
                kv = jnp.full((16,), k, jnp.int32)
                va = plsc.load_gather(ra, [rr, kv])
                vb = plsc.load_gather(rb, [rr, kv])
                df = va - vb
                ssum = ssum + df * df
            dd = _sqrt16(ssum)
            plsc.addupdate(accn_v.at[pl.ds(0, 16)],
                           jnp.maximum(_fsplat(DELTA) - dd, 0.0))
            return carry2
        lax.fori_loop(0, CP // 16, _grp, 0)

    _phase(ppa_hbm, ppb_hbm, _compute_pos)
    _phase(nga_hbm, ngb_hbm, _compute_neg)

    pltpu.sync_copy(accp_v, pos_out.at[pl.ds(wid * 16, 16)])
    pltpu.sync_copy(accn_v, neg_out.at[pl.ds(wid * 16, 16)])


def _seg_body(dist_hbm, ang_hbm, epar_hbm, gt_out,
              eidx_p, dist_v, ang_v, acc_v, sem):
    c = lax.axis_index("c")
    s = lax.axis_index("s")
    z16 = _fsplat(0.0)

    def _zero(i, carry):
        acc_v[pl.ds(i * 16, 16)] = z16
        acc_v[pl.ds(NPAD + i * 16, 16)] = z16
        acc_v[pl.ds(2 * NPAD + i * 16, 16)] = z16
        return carry
    lax.fori_loop(0, NPAD // 16, _zero, 0)

    slot0 = s * FS
    ebase = c * EPC

    def _chunk(i, carry):
        base = ebase + i * C2
        cp1 = pltpu.async_copy(epar_hbm.at[pl.ds(base, C2)], eidx_p, sem)
        cp2 = pltpu.async_copy(dist_hbm.at[pl.ds(base, C2)], dist_v, sem)
        cp3 = pltpu.async_copy(ang_hbm.at[pl.ds(base, C2)], ang_v, sem)
        cp1.wait()
        cp2.wait()
        cp3.wait()

        def _group(g, carry2):
            pi = eidx_p[pl.ds(g * 16, 16)]
            dv = dist_v[pl.ds(g * 16, 16)]
            av = ang_v[pl.ds(g * 16, 16)]
            for j in range(FS):
                slot = slot0 + j
                is_d = jnp.full((16,), slot, jnp.int32) < NUM_RBF
                val = jnp.where(is_d, dv, av)
                cd = (10.0 / (NUM_RBF - 1)) * slot.astype(jnp.float32)
                ca = (_PI / (NUM_RBF - 1)) * (slot - NUM_RBF).astype(jnp.float32)
                ctr = jnp.where(is_d, jnp.full((16,), 1.0, jnp.float32) * cd,
                                jnp.full((16,), 1.0, jnp.float32) * ca)
                dlt = val - ctr
                ev = jnp.exp(-(dlt * dlt))
                plsc.addupdate_scatter(acc_v.at[pl.ds(0, FS * NPAD)],
                                       [pi + j * NPAD], ev)
            return carry2
        lax.fori_loop(0, C2 // 16, _group, 0)
        return carry
    lax.fori_loop(0, EPC // C2, _chunk, 0)

    out_base = ((c * NS + s) * FS) * NPAD
    pltpu.sync_copy(acc_v, gt_out.at[pl.ds(out_base, FS * NPAD)])


def _tc_body(gt_ref, pos_ref, neg_ref, b1_ref, w2_ref, b2_ref, w3t_ref,
             b3t_ref, o_ref):
    gt_t = gt_ref[0, :NUM_RBF * 2, :N] + gt_ref[1, :NUM_RBF * 2, :N]
    ssum = jnp.sum(jnp.abs(gt_t), axis=0, keepdims=True)
    gtn = gt_t / jnp.maximum(ssum, 1e-12)
    h1 = jnp.maximum(b1_ref[...], 0.0)
    h2 = jnp.maximum(jnp.dot(h1, w2_ref[...],
                             preferred_element_type=jnp.float32)
                     + b2_ref[...], 0.0)
    pr = jnp.dot(w3t_ref[...], h2.reshape(-1, 1),
                 preferred_element_type=jnp.float32) + b3t_ref[...]
    p = pr / jnp.maximum(jnp.sum(jnp.abs(pr)), 1e-12)
    emd = jnp.sum(jnp.abs(p - gtn)) / (N * NUM_RBF * 2)
    pos_loss = jnp.sum(pos_ref[...]) / P
    neg_loss = jnp.sum(neg_ref[...]) / P
    o_ref[...] = jnp.reshape(emd + pos_loss + neg_loss, (1, 1))


def kernel(x, pos, batch, edge_index_3rd, parent_child_pairs, negative_pairs,
           edge_index, W1, b1, W2, b2, W3, b3):
    posx = pos[:, 0]
    posy = pos[:, 1]
    posz = pos[:, 2]
    epar = edge_index[0]
    echi = edge_index[1]
    ppa = parent_child_pairs[:, 0]
    ppb = parent_child_pairs[:, 1]
    nga = negative_pairs[:, 0]
    ngb = negative_pairs[:, 1]

    mesh = plsc.VectorSubcoreMesh(core_axis_name="c", subcore_axis_name="s")
    params = pltpu.CompilerParams(needs_layout_passes=False)

    geom = pl.kernel(
        _geom_body,
        out_type=[
            jax.ShapeDtypeStruct((E,), jnp.float32),
            jax.ShapeDtypeStruct((E,), jnp.float32),
        ],
        mesh=mesh,
        compiler_params=params,
        scratch_types=[
            pltpu.VMEM((N,), jnp.float32),
            pltpu.VMEM((N,), jnp.float32),
            pltpu.VMEM((N,), jnp.float32),
            pltpu.VMEM((CE,), jnp.int32),
            pltpu.VMEM((CE,), jnp.int32),
            pltpu.VMEM((CE,), jnp.float32),
            pltpu.VMEM((CE,), jnp.float32),
        ],
    )
    dist_e, ang_e = geom(posx, posy, posz, epar, echi)

    pairs = pl.kernel(
        _pairs_body,
        out_type=[
            jax.ShapeDtypeStruct((NW * 16,), jnp.float32),
            jax.ShapeDtypeStruct((NW * 16,), jnp.float32),
        ],
        mesh=mesh,
        compiler_params=params,
        scratch_types=[
            pltpu.VMEM((PPT,), jnp.int32),
            pltpu.VMEM((PPT,), jnp.int32),
            pltpu.VMEM((CP, D), jnp.float32),
            pltpu.VMEM((CP, D), jnp.float32),
            pltpu.VMEM((CP, D), jnp.float32),
            pltpu.VMEM((CP, D), jnp.float32),
            pltpu.VMEM((16,), jnp.float32),
            pltpu.VMEM((16,), jnp.float32),
            pltpu.SemaphoreType.DMA,
        ],
    )
    pos_parts, neg_parts = pairs(x, ppa, ppb, nga, ngb)

    seg = pl.kernel(
        _seg_body,
        out_type=jax.ShapeDtypeStruct((NC * NS * FS * NPAD,), jnp.float32),
        mesh=mesh,
        compiler_params=params,
        scratch_types=[
            pltpu.VMEM((C2,), jnp.int32),
            pltpu.VMEM((C2,), jnp.float32),
            pltpu.VMEM((C2,), jnp.float32),
            pltpu.VMEM((FS * NPAD,), jnp.float32),
            pltpu.SemaphoreType.DMA,
        ],
    )
    gt_parts = seg(dist_e, ang_e, epar)

    out = pl.pallas_call(
        _tc_body,
        out_shape=jax.ShapeDtypeStruct((1, 1), jnp.float32),
    )(gt_parts.reshape(NC, NS * FS, NPAD), pos_parts.reshape(4, 128),
      neg_parts.reshape(4, 128),
      b1.reshape(1, -1), W2, b2.reshape(1, -1), W3.T, b3.reshape(-1, 1))
    return out.reshape(())
